# baseline (device time: 8262 ns/iter reference)
import jax
import jax.numpy as jnp
from jax import lax
from jax.experimental import pallas as pl
from jax.experimental.pallas import tpu as pltpu

N_DEV = 8
HALO = 3
HPAD = 8


def kernel(x, k):
    b, s, c = x.shape

    def body(
        x_hbm, k_hbm, out_hbm,
        xv, kv, ov, send_ref, recv_ref,
        send_sem, recv_sem, cp_sems,
    ):
        my_i = lax.axis_index("i")
        left = (my_i - 1) % N_DEV
        right = (my_i + 1) % N_DEV

        credit_sem = pltpu.get_barrier_semaphore()

        @pl.when(my_i > 0)
        def _():
            pl.semaphore_signal(
                credit_sem, inc=1,
                device_id=(left,), device_id_type=pl.DeviceIdType.MESH,
            )

        @pl.when(my_i == 0)
        def _():
            recv_ref[...] = jnp.zeros((b, HPAD, c), jnp.float32)

        halo_cp = pltpu.make_async_copy(
            x_hbm.at[:, s - HPAD:, :], send_ref, cp_sems.at[0]
        )
        x_cp = pltpu.make_async_copy(x_hbm, xv, cp_sems.at[1])
        k_cp = pltpu.make_async_copy(k_hbm, kv, cp_sems.at[2])
        halo_cp.start()
        x_cp.start()
        k_cp.start()

        rdma = pltpu.make_async_remote_copy(
            src_ref=send_ref,
            dst_ref=recv_ref,
            send_sem=send_sem,
            recv_sem=recv_sem,
            device_id=(right,),
            device_id_type=pl.DeviceIdType.MESH,
        )

        halo_cp.wait()

        @pl.when(my_i < N_DEV - 1)
        def _():
            pl.semaphore_wait(credit_sem, 1)
            rdma.start()

        x_cp.wait()
        k_cp.wait()

        x_val = xv[...]
        acc = x_val[:, 0:s - HALO, :] * kv[0, :]
        acc = acc + x_val[:, 1:s - 2, :] * kv[1, :]
        acc = acc + x_val[:, 2:s - 1, :] * kv[2, :]
        acc = acc + x_val[:, HALO:, :] * kv[3, :]
        ov[:, HALO:, :] = acc * (1.0 / (1.0 + jnp.exp(-acc)))

        main_out_cp = pltpu.make_async_copy(
            ov.at[:, HPAD:, :], out_hbm.at[:, HPAD:, :], cp_sems.at[1]
        )
        main_out_cp.start()

        @pl.when(my_i > 0)
        def _():
            rdma.wait_recv()

        h = recv_ref[:, HPAD - HALO:, :]
        xw = jnp.concatenate([h, x_val[:, 0:HALO, :]], axis=1)
        head = xw[:, 0:HALO, :] * kv[0, :]
        head = head + xw[:, 1:HALO + 1, :] * kv[1, :]
        head = head + xw[:, 2:HALO + 2, :] * kv[2, :]
        head = head + xw[:, HALO:2 * HALO, :] * kv[3, :]
        ov[:, 0:HALO, :] = head * (1.0 / (1.0 + jnp.exp(-head)))

        head_out_cp = pltpu.make_async_copy(
            ov.at[:, 0:HPAD, :], out_hbm.at[:, 0:HPAD, :], cp_sems.at[0]
        )
        head_out_cp.start()

        main_out_cp.wait()
        head_out_cp.wait()

        @pl.when(my_i < N_DEV - 1)
        def _():
            rdma.wait_send()

    return pl.pallas_call(
        body,
        out_shape=jax.ShapeDtypeStruct((b, s, c), jnp.float32),
        in_specs=[
            pl.BlockSpec(memory_space=pl.ANY),
            pl.BlockSpec(memory_space=pl.ANY),
        ],
        out_specs=pl.BlockSpec(memory_space=pl.ANY),
        scratch_shapes=[
            pltpu.VMEM((b, s, c), jnp.float32),
            pltpu.VMEM((4, c), jnp.float32),
            pltpu.VMEM((b, s, c), jnp.float32),
            pltpu.VMEM((b, HPAD, c), jnp.float32),
            pltpu.VMEM((b, HPAD, c), jnp.float32),
            pltpu.SemaphoreType.DMA,
            pltpu.SemaphoreType.DMA,
            pltpu.SemaphoreType.DMA((3,)),
        ],
        compiler_params=pltpu.CompilerParams(collective_id=0),
    )(x, k)


# device time: 6328 ns/iter; 1.3056x vs baseline; 1.3056x over previous
import jax
import jax.numpy as jnp
from jax import lax
from jax.experimental import pallas as pl
from jax.experimental.pallas import tpu as pltpu

N_DEV = 8
HALO = 3
HPAD = 8


def kernel(x, k):
    b, s, c = x.shape

    def body(
        x_hbm, k_hbm, out_hbm,
        xv, kv, ov, send_ref, recv_ref,
        send_sem, recv_sem, cp_sems,
    ):
        my_i = lax.axis_index("i")
        left = (my_i - 1) % N_DEV
        right = (my_i + 1) % N_DEV

        credit_sem = pltpu.get_barrier_semaphore()

        @pl.when(my_i > 0)
        def _():
            pl.semaphore_signal(
                credit_sem, inc=1,
                device_id=(left,), device_id_type=pl.DeviceIdType.MESH,
            )

        @pl.when(my_i == 0)
        def _():
            recv_ref[...] = jnp.zeros((b, HPAD, c), jnp.float32)

        halo_cp = pltpu.make_async_copy(
            x_hbm.at[:, s - HPAD:, :], send_ref, cp_sems.at[0]
        )
        x_cp = pltpu.make_async_copy(x_hbm, xv, cp_sems.at[1])
        k_cp = pltpu.make_async_copy(k_hbm, kv, cp_sems.at[2])
        halo_cp.start()
        x_cp.start()
        k_cp.start()

        rdma = pltpu.make_async_remote_copy(
            src_ref=send_ref,
            dst_ref=recv_ref,
            send_sem=send_sem,
            recv_sem=recv_sem,
            device_id=(right,),
            device_id_type=pl.DeviceIdType.MESH,
        )

        halo_cp.wait()

        @pl.when(my_i < N_DEV - 1)
        def _():
            pl.semaphore_wait(credit_sem, 1)
            rdma.start()

        x_cp.wait()
        k_cp.wait()

        x_val = xv[...]
        acc = x_val[:, 0:s - HALO, :] * kv[0, :]
        acc = acc + x_val[:, 1:s - 2, :] * kv[1, :]
        acc = acc + x_val[:, 2:s - 1, :] * kv[2, :]
        acc = acc + x_val[:, HALO:, :] * kv[3, :]
        ov[:, HALO:, :] = acc * (1.0 / (1.0 + jnp.exp(-acc)))

        main_out_cp = pltpu.make_async_copy(
            ov.at[:, HPAD:, :], out_hbm.at[:, HPAD:, :], cp_sems.at[1]
        )
        main_out_cp.start()

        @pl.when(my_i > 0)
        def _():
            rdma.wait_recv()

        h = recv_ref[:, HPAD - HALO:, :]
        xw = jnp.concatenate([h, x_val[:, 0:HALO, :]], axis=1)
        head = xw[:, 0:HALO, :] * kv[0, :]
        head = head + xw[:, 1:HALO + 1, :] * kv[1, :]
        head = head + xw[:, 2:HALO + 2, :] * kv[2, :]
        head = head + xw[:, HALO:2 * HALO, :] * kv[3, :]
        ov[:, 0:HALO, :] = head * (1.0 / (1.0 + jnp.exp(-head)))

        head_out_cp = pltpu.make_async_copy(
            ov.at[:, 0:HPAD, :], out_hbm.at[:, 0:HPAD, :], cp_sems.at[0]
        )
        head_out_cp.start()

        main_out_cp.wait()
        head_out_cp.wait()

        @pl.when(my_i < N_DEV - 1)
        def _():
            rdma.wait_send()

    return pl.pallas_call(
        body,
        out_shape=jax.ShapeDtypeStruct((b, s, c), jnp.float32),
        in_specs=[
            pl.BlockSpec(memory_space=pltpu.HBM),
            pl.BlockSpec(memory_space=pltpu.HBM),
        ],
        out_specs=pl.BlockSpec(memory_space=pltpu.HBM),
        scratch_shapes=[
            pltpu.VMEM((b, s, c), jnp.float32),
            pltpu.VMEM((4, c), jnp.float32),
            pltpu.VMEM((b, s, c), jnp.float32),
            pltpu.VMEM((b, HPAD, c), jnp.float32),
            pltpu.VMEM((b, HPAD, c), jnp.float32),
            pltpu.SemaphoreType.DMA,
            pltpu.SemaphoreType.DMA,
            pltpu.SemaphoreType.DMA((3,)),
        ],
        compiler_params=pltpu.CompilerParams(collective_id=0),
    )(x, k)


# device time: 5998 ns/iter; 1.3775x vs baseline; 1.0550x over previous
import jax
import jax.numpy as jnp
from jax import lax
from jax.experimental import pallas as pl
from jax.experimental.pallas import tpu as pltpu

N_DEV = 8
HALO = 3
HPAD = 8


def kernel(x, k):
    b, s, c = x.shape

    def body(
        x_hbm, k_hbm, out_hbm,
        xv, kv, ov, send_ref, recv_ref,
        send_sem, recv_sem, cp_sems,
    ):
        my_i = lax.axis_index("i")
        left = (my_i - 1) % N_DEV
        right = (my_i + 1) % N_DEV

        credit_sem = pltpu.get_barrier_semaphore()

        @pl.when(my_i > 0)
        def _():
            pl.semaphore_signal(
                credit_sem, inc=1,
                device_id=(left,), device_id_type=pl.DeviceIdType.MESH,
            )

        @pl.when(my_i == 0)
        def _():
            recv_ref[...] = jnp.zeros((b, HPAD, c), jnp.float32)

        halo_cp = pltpu.make_async_copy(
            x_hbm.at[:, s - HPAD:, :], send_ref, cp_sems.at[0]
        )
        x_cp = pltpu.make_async_copy(x_hbm, xv, cp_sems.at[1])
        k_cp = pltpu.make_async_copy(k_hbm, kv, cp_sems.at[2])
        halo_cp.start()
        x_cp.start()
        k_cp.start()

        rdma = pltpu.make_async_remote_copy(
            src_ref=send_ref,
            dst_ref=recv_ref,
            send_sem=send_sem,
            recv_sem=recv_sem,
            device_id=(right,),
            device_id_type=pl.DeviceIdType.MESH,
        )

        halo_cp.wait()

        @pl.when(my_i < N_DEV - 1)
        def _():
            pl.semaphore_wait(credit_sem, 1)
            rdma.start()

        x_cp.wait()
        k_cp.wait()

        x_val = xv[...]
        acc = x_val[:, 0:s - HALO, :] * kv[0, :]
        acc = acc + x_val[:, 1:s - 2, :] * kv[1, :]
        acc = acc + x_val[:, 2:s - 1, :] * kv[2, :]
        acc = acc + x_val[:, HALO:, :] * kv[3, :]
        ov[:, HALO:, :] = acc * (1.0 / (1.0 + jnp.exp(-acc)))

        main_out_cp = pltpu.make_async_copy(
            ov.at[:, HPAD:, :], out_hbm.at[:, HPAD:, :], cp_sems.at[1]
        )
        main_out_cp.start()

        @pl.when(my_i > 0)
        def _():
            rdma.wait_recv()

        h = recv_ref[:, HPAD - HALO:, :]
        xw = jnp.concatenate([h, x_val[:, 0:HALO, :]], axis=1)
        head = xw[:, 0:HALO, :] * kv[0, :]
        head = head + xw[:, 1:HALO + 1, :] * kv[1, :]
        head = head + xw[:, 2:HALO + 2, :] * kv[2, :]
        head = head + xw[:, HALO:2 * HALO, :] * kv[3, :]
        ov[:, 0:HALO, :] = head * (1.0 / (1.0 + jnp.exp(-head)))

        head_out_cp = pltpu.make_async_copy(
            ov.at[:, 0:HPAD, :], out_hbm.at[:, 0:HPAD, :], cp_sems.at[0]
        )
        head_out_cp.start()

        main_out_cp.wait()
        head_out_cp.wait()

        @pl.when(my_i < N_DEV - 1)
        def _():
            rdma.wait_send()

    out = pl.pallas_call(
        body,
        out_shape=jax.ShapeDtypeStruct((b, s, c), jnp.float32),
        in_specs=[
            pl.BlockSpec(memory_space=pltpu.HBM),
            pl.BlockSpec(memory_space=pltpu.HBM),
        ],
        out_specs=pl.BlockSpec(memory_space=pltpu.HBM),
        scratch_shapes=[
            pltpu.VMEM((b, s, c), jnp.float32),
            pltpu.VMEM((4, c), jnp.float32),
            pltpu.VMEM((b, s, c), jnp.float32),
            pltpu.VMEM((b, HPAD, c), jnp.float32),
            pltpu.VMEM((b, HPAD, c), jnp.float32),
            pltpu.SemaphoreType.DMA,
            pltpu.SemaphoreType.DMA,
            pltpu.SemaphoreType.DMA((3,)),
        ],
        compiler_params=pltpu.CompilerParams(collective_id=0),
    )(
        pltpu.with_memory_space_constraint(x, pltpu.HBM),
        pltpu.with_memory_space_constraint(k, pltpu.HBM),
    )
    return pltpu.with_memory_space_constraint(out, pltpu.HBM)


# device time: 5694 ns/iter; 1.4510x vs baseline; 1.0534x over previous
import jax
import jax.numpy as jnp
from jax import lax
from jax.experimental import pallas as pl
from jax.experimental.pallas import tpu as pltpu

N_DEV = 8
HALO = 3
HPAD = 8


def kernel(x, k):
    b, s, c = x.shape

    def body(
        x_hbm, k_hbm, o_ref,
        xv, kv, send_ref, recv_ref,
        send_sem, recv_sem, cp_sems,
    ):
        my_i = lax.axis_index("i")
        left = (my_i - 1) % N_DEV
        right = (my_i + 1) % N_DEV

        credit_sem = pltpu.get_barrier_semaphore()

        @pl.when(my_i > 0)
        def _():
            pl.semaphore_signal(
                credit_sem, inc=1,
                device_id=(left,), device_id_type=pl.DeviceIdType.MESH,
            )

        @pl.when(my_i == 0)
        def _():
            recv_ref[...] = jnp.zeros((b, HPAD, c), jnp.float32)

        halo_cp = pltpu.make_async_copy(
            x_hbm.at[:, s - HPAD:, :], send_ref, cp_sems.at[0]
        )
        x_cp = pltpu.make_async_copy(x_hbm, xv, cp_sems.at[1])
        k_cp = pltpu.make_async_copy(k_hbm, kv, cp_sems.at[2])
        halo_cp.start()
        x_cp.start()
        k_cp.start()

        rdma = pltpu.make_async_remote_copy(
            src_ref=send_ref,
            dst_ref=recv_ref,
            send_sem=send_sem,
            recv_sem=recv_sem,
            device_id=(right,),
            device_id_type=pl.DeviceIdType.MESH,
        )

        halo_cp.wait()

        @pl.when(my_i < N_DEV - 1)
        def _():
            pl.semaphore_wait(credit_sem, 1)
            rdma.start()

        x_cp.wait()
        k_cp.wait()

        x_val = xv[...]
        acc = x_val[:, 0:s - HALO, :] * kv[0, :]
        acc = acc + x_val[:, 1:s - 2, :] * kv[1, :]
        acc = acc + x_val[:, 2:s - 1, :] * kv[2, :]
        acc = acc + x_val[:, HALO:, :] * kv[3, :]
        o_ref[:, HALO:, :] = acc * jax.nn.sigmoid(acc)

        @pl.when(my_i > 0)
        def _():
            rdma.wait_recv()

        h = recv_ref[:, HPAD - HALO:, :]
        xw = jnp.concatenate([h, x_val[:, 0:HALO, :]], axis=1)
        head = xw[:, 0:HALO, :] * kv[0, :]
        head = head + xw[:, 1:HALO + 1, :] * kv[1, :]
        head = head + xw[:, 2:HALO + 2, :] * kv[2, :]
        head = head + xw[:, HALO:2 * HALO, :] * kv[3, :]
        o_ref[:, 0:HALO, :] = head * jax.nn.sigmoid(head)

        @pl.when(my_i < N_DEV - 1)
        def _():
            rdma.wait_send()

    out = pl.pallas_call(
        body,
        out_shape=jax.ShapeDtypeStruct((b, s, c), jnp.float32),
        in_specs=[
            pl.BlockSpec(memory_space=pltpu.HBM),
            pl.BlockSpec(memory_space=pltpu.HBM),
        ],
        out_specs=pl.BlockSpec(memory_space=pltpu.VMEM),
        scratch_shapes=[
            pltpu.VMEM((b, s, c), jnp.float32),
            pltpu.VMEM((4, c), jnp.float32),
            pltpu.VMEM((b, HPAD, c), jnp.float32),
            pltpu.VMEM((b, HPAD, c), jnp.float32),
            pltpu.SemaphoreType.DMA,
            pltpu.SemaphoreType.DMA,
            pltpu.SemaphoreType.DMA((3,)),
        ],
        compiler_params=pltpu.CompilerParams(collective_id=0),
    )(
        pltpu.with_memory_space_constraint(x, pltpu.HBM),
        pltpu.with_memory_space_constraint(k, pltpu.HBM),
    )
    return out
